# Initial kernel scaffold; baseline (speedup 1.0000x reference)
import jax
import jax.numpy as jnp
from jax.experimental import pallas as pl


def kernel(dnn_feat, seq_feat, seq_mask, emb0, emb_rest, fc1_W, fc1_b, fc2_W, fc2_b, ck, cb, d1_W, d1_b, d2_W, d2_b, lin_W, lin_b):
    raise NotImplementedError("write your pallas kernel here")


# speedup vs baseline: 2.6992x; 2.6992x over previous
import functools

import jax
import jax.numpy as jnp
from jax import lax
from jax.experimental import pallas as pl
from jax.experimental.pallas import tpu as pltpu
from jax.experimental.pallas import tpu_sc as plsc

B = 1024
F = 26
L = 50
D = 64
NF = F - 1
VREST = 1001
TOTX = F * D + D
BB = 128
NC = 2
NS = 16
NW = NC * NS

PW_E0 = B // NW
PW_ER = (B * NF) // NW
PW_SEQ = (B * L) // NW
CH = 80


def _gather_body(emb0_hbm, embr_hbm, idx0_hbm, idxr_hbm, idxs_hbm,
                 e0_out, er_out, seq_out, idx_v, rows_v, sem):
    wid = lax.axis_index("s") * NC + lax.axis_index("c")

    def phase(table, idxh, outh, per_w):
        base = wid * per_w
        pltpu.sync_copy(idxh.at[pl.ds(base, per_w)], idx_v.at[pl.ds(0, per_w)])
        nch = per_w // CH
        cps = []
        if nch == 0:
            cps.append(pltpu.async_copy(
                table.at[idx_v.at[pl.ds(0, per_w)]],
                rows_v.at[pl.ds(0, per_w)], sem))
        else:
            for j in range(nch):
                cps.append(pltpu.async_copy(
                    table.at[idx_v.at[pl.ds(j * CH, CH)]],
                    rows_v.at[pl.ds(j * CH, CH)], sem))
        for cp in cps:
            cp.wait()
        pltpu.sync_copy(rows_v.at[pl.ds(0, per_w)], outh.at[pl.ds(base, per_w)])

    phase(emb0_hbm, idx0_hbm, e0_out, PW_E0)
    phase(embr_hbm, idxr_hbm, er_out, PW_ER)
    phase(emb0_hbm, idxs_hbm, seq_out, PW_SEQ)


def _sc_gather(emb0, embr, idx0, idxr, idxs):
    mesh = plsc.VectorSubcoreMesh(core_axis_name="c", subcore_axis_name="s")
    k = functools.partial(
        pl.kernel,
        mesh=mesh,
        compiler_params=pltpu.CompilerParams(use_tc_tiling_on_sc=False),
        out_type=[
            jax.ShapeDtypeStruct((B, D), jnp.float32),
            jax.ShapeDtypeStruct((B * NF, D), jnp.float32),
            jax.ShapeDtypeStruct((B * L, D), jnp.float32),
        ],
        scratch_types=[
            pltpu.VMEM((PW_SEQ,), jnp.int32),
            pltpu.VMEM((PW_SEQ, D), jnp.float32),
            pltpu.SemaphoreType.DMA,
        ],
    )(_gather_body)
    return k(emb0, embr, idx0, idxr, idxs)


def _dense_body(e0_r, er_r, seq_r, mask_r, w128_r, wtd_r, fc1b_r, v2_r,
                fc2b_r, ck0_r, cb0_r, ck1_r, cb1_r, d1w_r, d1b_r, d2w_r,
                d2b_r, linx_r, lind_r, linb_r, out_r):
    f32 = jnp.float32
    e0 = e0_r[...]
    erf = er_r[...]
    seq = seq_r[...]
    target = jnp.concatenate([e0, erf[:, : 3 * D]], axis=1)
    c = jnp.dot(target, wtd_r[...], preferred_element_type=f32) + fc1b_r[...]

    seq2 = seq.reshape(BB * L, D)
    su = (seq * e0[:, None, :]).reshape(BB * L, D)
    hm = (jnp.dot(seq2, w128_r[...][:D], preferred_element_type=f32)
          + jnp.dot(su, w128_r[...][D:], preferred_element_type=f32))
    h = jnp.maximum(hm.reshape(BB, L, 4 * D) + c[:, None, :], 0.0)
    aw = (h * v2_r[...][0]).sum(-1) + fc2b_r[...]
    aw = jnp.where(mask_r[...] < 0.5, -1e9, aw)
    m = jnp.max(aw, axis=1, keepdims=True)
    ex = jnp.exp(aw - m)
    p = ex / jnp.sum(ex, axis=1, keepdims=True)
    din = (p[:, :, None] * seq).sum(1)

    tx = jnp.concatenate([e0, erf, din], axis=1)
    w0 = jnp.dot(tx, ck0_r[...], preferred_element_type=f32) + cb0_r[...]
    x1 = tx * w0 + tx
    w1 = jnp.dot(x1, ck1_r[...], preferred_element_type=f32) + cb1_r[...]
    x2 = tx * w1 + x1
    dh = jnp.maximum(jnp.dot(tx, d1w_r[...], preferred_element_type=f32)
                     + d1b_r[...], 0.0)
    dh2 = jnp.maximum(jnp.dot(dh, d2w_r[...], preferred_element_type=f32)
                      + d2b_r[...], 0.0)
    out = ((x2 * linx_r[...]).sum(-1, keepdims=True)
           + (dh2 * lind_r[...]).sum(-1, keepdims=True) + linb_r[...])
    out_r[...] = out


def _dense_specs():
    def blk(shape, imap):
        return pl.BlockSpec(shape, imap)

    bi = lambda i: (i, 0)
    w2 = lambda i: (0, 0)
    in_specs = [
        blk((BB, D), bi),
        blk((BB, NF * D), bi),
        pl.BlockSpec((BB, L, D), lambda i: (i, 0, 0)),
        blk((BB, L), bi),
        blk((2 * D, 4 * D), w2),
        blk((4 * D, 4 * D), w2),
        blk((1, 4 * D), w2),
        blk((1, 4 * D), w2),
        blk((1, 1), w2),
        blk((TOTX, TOTX), w2),
        blk((1, TOTX), w2),
        blk((TOTX, TOTX), w2),
        blk((1, TOTX), w2),
        blk((TOTX, 1024), w2),
        blk((1, 1024), w2),
        blk((1024, 512), w2),
        blk((1, 512), w2),
        blk((1, TOTX), w2),
        blk((1, 512), w2),
        blk((1, 1), w2),
    ]
    out_spec = blk((BB, 1), bi)
    return in_specs, out_spec


def _dense(args):
    in_specs, out_spec = _dense_specs()
    return pl.pallas_call(
        _dense_body,
        grid=(B // BB,),
        in_specs=in_specs,
        out_specs=out_spec,
        out_shape=jax.ShapeDtypeStruct((B, 1), jnp.float32),
    )(*args)


def kernel(dnn_feat, seq_feat, seq_mask, emb0, emb_rest, fc1_W, fc1_b, fc2_W,
           fc2_b, ck, cb, d1_W, d1_b, d2_W, d2_b, lin_W, lin_b):
    dnn_feat = dnn_feat.astype(jnp.int32)
    seq_feat = seq_feat.astype(jnp.int32)
    embr = emb_rest.reshape(NF * VREST, D)
    idx0 = dnn_feat[:, 0]
    offs = (jnp.arange(NF, dtype=jnp.int32) * VREST)[None, :]
    idxr = (dnn_feat[:, 1:] + offs).reshape(-1)
    idxs = seq_feat.reshape(-1)

    e0_g, er_g, seq_g = _sc_gather(emb0, embr, idx0, idxr, idxs)
    er_g = er_g.reshape(B, NF * D)
    seq_g = seq_g.reshape(B, L, D)

    wt = fc1_W.T
    w128 = jnp.concatenate([wt[0:D] + wt[8 * D:9 * D], wt[12 * D:13 * D]],
                           axis=0)
    wtd = wt[4 * D:8 * D] - wt[8 * D:12 * D]
    args = (
        e0_g, er_g, seq_g, seq_mask,
        w128, wtd, fc1_b.reshape(1, -1), fc2_W.reshape(1, -1),
        fc2_b.reshape(1, 1),
        ck[0][:TOTX, :TOTX], cb[0][:TOTX].reshape(1, -1),
        ck[1][:TOTX, :TOTX], cb[1][:TOTX].reshape(1, -1),
        d1_W[:, :TOTX].T, d1_b.reshape(1, -1),
        d2_W.T, d2_b.reshape(1, -1),
        lin_W[:, :TOTX], lin_W[:, F * D + 4 * D:].reshape(1, -1),
        lin_b.reshape(1, 1),
    )
    return _dense(args)


# speedup vs baseline: 3.0040x; 1.1129x over previous
import functools

import jax
import jax.numpy as jnp
from jax import lax
from jax.experimental import pallas as pl
from jax.experimental.pallas import tpu as pltpu
from jax.experimental.pallas import tpu_sc as plsc

B = 1024
F = 26
L = 50
D = 64
NF = F - 1
VREST = 1001
TOTX = F * D + D
BB = 256
NC = 2
NS = 16
NW = NC * NS

PW_E0 = B // NW
PW_ER = (B * NF) // NW
PW_SEQ = (B * L) // NW
CH = 80


def _gather_body(emb0_hbm, embr_hbm, idx0_hbm, idxr_hbm, idxs_hbm,
                 e0_out, er_out, seq_out, idx_v, rows_v, sem):
    wid = lax.axis_index("s") * NC + lax.axis_index("c")

    def phase(table, idxh, outh, per_w):
        base = wid * per_w
        pltpu.sync_copy(idxh.at[pl.ds(base, per_w)], idx_v.at[pl.ds(0, per_w)])
        nch = per_w // CH
        cps = []
        if nch == 0:
            cps.append(pltpu.async_copy(
                table.at[idx_v.at[pl.ds(0, per_w)]],
                rows_v.at[pl.ds(0, per_w)], sem))
        else:
            for j in range(nch):
                cps.append(pltpu.async_copy(
                    table.at[idx_v.at[pl.ds(j * CH, CH)]],
                    rows_v.at[pl.ds(j * CH, CH)], sem))
        for cp in cps:
            cp.wait()
        pltpu.sync_copy(rows_v.at[pl.ds(0, per_w)], outh.at[pl.ds(base, per_w)])

    phase(emb0_hbm, idx0_hbm, e0_out, PW_E0)
    phase(embr_hbm, idxr_hbm, er_out, PW_ER)
    phase(emb0_hbm, idxs_hbm, seq_out, PW_SEQ)


def _sc_gather(emb0, embr, idx0, idxr, idxs):
    mesh = plsc.VectorSubcoreMesh(core_axis_name="c", subcore_axis_name="s")
    k = functools.partial(
        pl.kernel,
        mesh=mesh,
        compiler_params=pltpu.CompilerParams(use_tc_tiling_on_sc=False),
        out_type=[
            jax.ShapeDtypeStruct((B, D), jnp.float32),
            jax.ShapeDtypeStruct((B * NF, D), jnp.float32),
            jax.ShapeDtypeStruct((B * L, D), jnp.float32),
        ],
        scratch_types=[
            pltpu.VMEM((PW_SEQ,), jnp.int32),
            pltpu.VMEM((PW_SEQ, D), jnp.float32),
            pltpu.SemaphoreType.DMA,
        ],
    )(_gather_body)
    return k(emb0, embr, idx0, idxr, idxs)


def _dense_body(e0_r, er_r, seq_r, mask_r, w128_r, wtd_r, fc1b_r, v2_r,
                fc2b_r, ck0_r, cb0_r, ck1_r, cb1_r, d1w_r, d1b_r, d2w_r,
                d2b_r, linx_r, lind_r, linb_r, out_r):
    f32 = jnp.float32
    bf16 = jnp.bfloat16
    e0 = e0_r[...]
    erf = er_r[...]
    seq = seq_r[...]
    target = jnp.concatenate([e0, erf[:, : 3 * D]], axis=1)
    c = jnp.dot(target.astype(bf16), wtd_r[...],
                preferred_element_type=f32) + fc1b_r[...]

    seq2 = seq.reshape(BB * L, D)
    su = (seq * e0[:, None, :]).reshape(BB * L, D)
    hm = (jnp.dot(seq2.astype(bf16), w128_r[...][:D],
                  preferred_element_type=f32)
          + jnp.dot(su.astype(bf16), w128_r[...][D:],
                    preferred_element_type=f32))
    h = jnp.maximum(hm.reshape(BB, L, 4 * D) + c[:, None, :], 0.0)
    aw = (h * v2_r[...][0]).sum(-1) + fc2b_r[...]
    aw = jnp.where(mask_r[...] < 0.5, -1e9, aw)
    m = jnp.max(aw, axis=1, keepdims=True)
    ex = jnp.exp(aw - m)
    p = ex / jnp.sum(ex, axis=1, keepdims=True)
    din = (p[:, :, None] * seq).sum(1)

    tx = jnp.concatenate([e0, erf, din], axis=1)
    txh = tx.astype(bf16)
    w0 = jnp.dot(txh, ck0_r[...], preferred_element_type=f32) + cb0_r[...]
    x1 = tx * w0 + tx
    w1 = jnp.dot(x1.astype(bf16), ck1_r[...],
                 preferred_element_type=f32) + cb1_r[...]
    x2 = tx * w1 + x1
    dh = jnp.maximum(jnp.dot(txh, d1w_r[...], preferred_element_type=f32)
                     + d1b_r[...], 0.0)
    dh2 = jnp.maximum(jnp.dot(dh.astype(bf16), d2w_r[...],
                              preferred_element_type=f32) + d2b_r[...], 0.0)
    out = ((x2 * linx_r[...]).sum(-1, keepdims=True)
           + (dh2 * lind_r[...]).sum(-1, keepdims=True) + linb_r[...])
    out_r[...] = out


def _dense_specs():
    def blk(shape, imap):
        return pl.BlockSpec(shape, imap)

    bi = lambda i: (i, 0)
    w2 = lambda i: (0, 0)
    in_specs = [
        blk((BB, D), bi),
        blk((BB, NF * D), bi),
        pl.BlockSpec((BB, L, D), lambda i: (i, 0, 0)),
        blk((BB, L), bi),
        blk((2 * D, 4 * D), w2),
        blk((4 * D, 4 * D), w2),
        blk((1, 4 * D), w2),
        blk((1, 4 * D), w2),
        blk((1, 1), w2),
        blk((TOTX, TOTX), w2),
        blk((1, TOTX), w2),
        blk((TOTX, TOTX), w2),
        blk((1, TOTX), w2),
        blk((TOTX, 1024), w2),
        blk((1, 1024), w2),
        blk((1024, 512), w2),
        blk((1, 512), w2),
        blk((1, TOTX), w2),
        blk((1, 512), w2),
        blk((1, 1), w2),
    ]
    out_spec = blk((BB, 1), bi)
    return in_specs, out_spec


def _dense(args):
    in_specs, out_spec = _dense_specs()
    return pl.pallas_call(
        _dense_body,
        grid=(B // BB,),
        in_specs=in_specs,
        out_specs=out_spec,
        out_shape=jax.ShapeDtypeStruct((B, 1), jnp.float32),
        compiler_params=pltpu.CompilerParams(
            vmem_limit_bytes=120 * 1024 * 1024),
    )(*args)


def kernel(dnn_feat, seq_feat, seq_mask, emb0, emb_rest, fc1_W, fc1_b, fc2_W,
           fc2_b, ck, cb, d1_W, d1_b, d2_W, d2_b, lin_W, lin_b):
    dnn_feat = dnn_feat.astype(jnp.int32)
    seq_feat = seq_feat.astype(jnp.int32)
    embr = emb_rest.reshape(NF * VREST, D)
    idx0 = dnn_feat[:, 0]
    offs = (jnp.arange(NF, dtype=jnp.int32) * VREST)[None, :]
    idxr = (dnn_feat[:, 1:] + offs).reshape(-1)
    idxs = seq_feat.reshape(-1)

    e0_g, er_g, seq_g = _sc_gather(emb0, embr, idx0, idxr, idxs)
    er_g = er_g.reshape(B, NF * D)
    seq_g = seq_g.reshape(B, L, D)

    wt = fc1_W.T
    bf16 = jnp.bfloat16
    w128 = jnp.concatenate([wt[0:D] + wt[8 * D:9 * D], wt[12 * D:13 * D]],
                           axis=0).astype(bf16)
    wtd = (wt[4 * D:8 * D] - wt[8 * D:12 * D]).astype(bf16)
    args = (
        e0_g, er_g, seq_g, seq_mask,
        w128, wtd, fc1_b.reshape(1, -1), fc2_W.reshape(1, -1),
        fc2_b.reshape(1, 1),
        ck[0][:TOTX, :TOTX].astype(bf16), cb[0][:TOTX].reshape(1, -1),
        ck[1][:TOTX, :TOTX].astype(bf16), cb[1][:TOTX].reshape(1, -1),
        d1_W[:, :TOTX].T.astype(bf16), d1_b.reshape(1, -1),
        d2_W.T.astype(bf16), d2_b.reshape(1, -1),
        lin_W[:, :TOTX], lin_W[:, F * D + 4 * D:].reshape(1, -1),
        lin_b.reshape(1, 1),
    )
    return _dense(args)


# speedup vs baseline: 3.0793x; 1.0251x over previous
import functools

import jax
import jax.numpy as jnp
from jax import lax
from jax.experimental import pallas as pl
from jax.experimental.pallas import tpu as pltpu
from jax.experimental.pallas import tpu_sc as plsc

B = 1024
F = 26
L = 50
D = 64
NF = F - 1
VREST = 1001
TOTX = F * D + D
TOTP = 1792
BB = 256
NC = 2
NS = 16
NW = NC * NS

PW_E0 = B // NW
PW_ER = (B * NF) // NW
PW_SEQ = (B * L) // NW
CH = 80


def _gather_body(emb0_hbm, embr_hbm, idx0_hbm, idxr_hbm, idxs_hbm,
                 e0_out, er_out, seq_out, idx_v, rows_v, sem):
    wid = lax.axis_index("s") * NC + lax.axis_index("c")

    def phase(table, idxh, outh, per_w):
        base = wid * per_w
        pltpu.sync_copy(idxh.at[pl.ds(base, per_w)], idx_v.at[pl.ds(0, per_w)])
        nch = per_w // CH
        cps = []
        if nch == 0:
            cps.append(pltpu.async_copy(
                table.at[idx_v.at[pl.ds(0, per_w)]],
                rows_v.at[pl.ds(0, per_w)], sem))
        else:
            for j in range(nch):
                cps.append(pltpu.async_copy(
                    table.at[idx_v.at[pl.ds(j * CH, CH)]],
                    rows_v.at[pl.ds(j * CH, CH)], sem))
        for cp in cps:
            cp.wait()
        pltpu.sync_copy(rows_v.at[pl.ds(0, per_w)], outh.at[pl.ds(base, per_w)])

    phase(emb0_hbm, idx0_hbm, e0_out, PW_E0)
    phase(embr_hbm, idxr_hbm, er_out, PW_ER)
    phase(emb0_hbm, idxs_hbm, seq_out, PW_SEQ)


def _sc_gather(emb0, embr, idx0, idxr, idxs):
    mesh = plsc.VectorSubcoreMesh(core_axis_name="c", subcore_axis_name="s")
    k = functools.partial(
        pl.kernel,
        mesh=mesh,
        compiler_params=pltpu.CompilerParams(use_tc_tiling_on_sc=False),
        out_type=[
            jax.ShapeDtypeStruct((B, D), jnp.float32),
            jax.ShapeDtypeStruct((B * NF, D), jnp.float32),
            jax.ShapeDtypeStruct((B * L, D), jnp.float32),
        ],
        scratch_types=[
            pltpu.VMEM((PW_SEQ,), jnp.int32),
            pltpu.VMEM((PW_SEQ, D), jnp.float32),
            pltpu.SemaphoreType.DMA,
        ],
    )(_gather_body)
    return k(emb0, embr, idx0, idxr, idxs)


def _din_body(e0_r, er_r, seq_r, mask_r, w128_r, wtd_r, fc1b_r, v2_r,
              fc2b_r, din_r):
    f32 = jnp.float32
    bf16 = jnp.bfloat16
    e0 = e0_r[...]
    seq = seq_r[...]
    target = jnp.concatenate([e0, er_r[...][:, : 3 * D]], axis=1)
    c = jnp.dot(target.astype(bf16), wtd_r[...],
                preferred_element_type=f32) + fc1b_r[...]

    seq2 = seq.reshape(BB * L, D)
    su = (seq * e0[:, None, :]).reshape(BB * L, D)
    hm = (jnp.dot(seq2.astype(bf16), w128_r[...][:D],
                  preferred_element_type=f32)
          + jnp.dot(su.astype(bf16), w128_r[...][D:],
                    preferred_element_type=f32))
    h = jnp.maximum(hm.reshape(BB, L, 4 * D) + c[:, None, :], 0.0)
    aw = (h * v2_r[...][0]).sum(-1) + fc2b_r[...]
    aw = jnp.where(mask_r[...] < 0.5, -1e9, aw)
    m = jnp.max(aw, axis=1, keepdims=True)
    ex = jnp.exp(aw - m)
    p = ex / jnp.sum(ex, axis=1, keepdims=True)
    din_r[...] = (p[:, :, None] * seq).sum(1)


def _din(e0_g, er_g, seq_g, seq_mask, w128, wtd, fc1b, v2, fc2b):
    bi = lambda i: (i, 0)
    w2 = lambda i: (0, 0)
    in_specs = [
        pl.BlockSpec((BB, D), bi),
        pl.BlockSpec((BB, 4 * D), bi),
        pl.BlockSpec((BB, L, D), lambda i: (i, 0, 0)),
        pl.BlockSpec((BB, L), bi),
        pl.BlockSpec((2 * D, 4 * D), w2),
        pl.BlockSpec((4 * D, 4 * D), w2),
        pl.BlockSpec((1, 4 * D), w2),
        pl.BlockSpec((1, 4 * D), w2),
        pl.BlockSpec((1, 1), w2),
    ]
    return pl.pallas_call(
        _din_body,
        grid=(B // BB,),
        in_specs=in_specs,
        out_specs=pl.BlockSpec((BB, D), bi),
        out_shape=jax.ShapeDtypeStruct((B, D), jnp.float32),
        compiler_params=pltpu.CompilerParams(
            vmem_limit_bytes=100 * 1024 * 1024),
    )(e0_g, er_g, seq_g, seq_mask, w128, wtd, fc1b, v2, fc2b)


def _cross_body(e0_r, er_r, din_r, ck_r, cb0_r, cb1_r, d1w_r, d1b_r, d2w_r,
                d2b_r, linx_r, lind_r, linb_r, out_r):
    f32 = jnp.float32
    bf16 = jnp.bfloat16
    zer = jnp.zeros((B, TOTP - TOTX), f32)
    tx = jnp.concatenate([e0_r[...], er_r[...], din_r[...], zer], axis=1)
    txh = tx.astype(bf16)
    w0 = jnp.dot(txh, ck_r[0], preferred_element_type=f32) + cb0_r[...]
    x1 = tx * w0 + tx
    w1 = jnp.dot(x1.astype(bf16), ck_r[1],
                 preferred_element_type=f32) + cb1_r[...]
    x2 = tx * w1 + x1
    dh = jnp.maximum(jnp.dot(txh, d1w_r[...], preferred_element_type=f32)
                     + d1b_r[...], 0.0)
    dh2 = jnp.maximum(jnp.dot(dh.astype(bf16), d2w_r[...],
                              preferred_element_type=f32) + d2b_r[...], 0.0)
    out = ((x2 * linx_r[...]).sum(-1, keepdims=True)
           + (dh2 * lind_r[...]).sum(-1, keepdims=True) + linb_r[...])
    out_r[...] = out


def _cross(e0_g, er_g, din, ckh, cb0, cb1, d1w, d1b, d2w, d2b, linx, lind,
           linb):
    w2 = lambda i: (0, 0)
    in_specs = [
        pl.BlockSpec((B, D), w2),
        pl.BlockSpec((B, NF * D), w2),
        pl.BlockSpec((B, D), w2),
        pl.BlockSpec((2, TOTP, TOTP), lambda i: (0, 0, 0)),
        pl.BlockSpec((1, TOTP), w2),
        pl.BlockSpec((1, TOTP), w2),
        pl.BlockSpec((TOTP, 1024), w2),
        pl.BlockSpec((1, 1024), w2),
        pl.BlockSpec((1024, 512), w2),
        pl.BlockSpec((1, 512), w2),
        pl.BlockSpec((1, TOTP), w2),
        pl.BlockSpec((1, 512), w2),
        pl.BlockSpec((1, 1), w2),
    ]
    return pl.pallas_call(
        _cross_body,
        grid=(1,),
        in_specs=in_specs,
        out_specs=pl.BlockSpec((B, 1), w2),
        out_shape=jax.ShapeDtypeStruct((B, 1), jnp.float32),
        compiler_params=pltpu.CompilerParams(
            vmem_limit_bytes=120 * 1024 * 1024),
    )(e0_g, er_g, din, ckh, cb0, cb1, d1w, d1b, d2w, d2b, linx, lind, linb)


def kernel(dnn_feat, seq_feat, seq_mask, emb0, emb_rest, fc1_W, fc1_b, fc2_W,
           fc2_b, ck, cb, d1_W, d1_b, d2_W, d2_b, lin_W, lin_b):
    dnn_feat = dnn_feat.astype(jnp.int32)
    seq_feat = seq_feat.astype(jnp.int32)
    embr = emb_rest.reshape(NF * VREST, D)
    idx0 = dnn_feat[:, 0]
    offs = (jnp.arange(NF, dtype=jnp.int32) * VREST)[None, :]
    idxr = (dnn_feat[:, 1:] + offs).reshape(-1)
    idxs = seq_feat.reshape(-1)

    e0_g, er_g, seq_g = _sc_gather(emb0, embr, idx0, idxr, idxs)
    er_g = er_g.reshape(B, NF * D)
    seq_g = seq_g.reshape(B, L, D)

    wt = fc1_W.T
    bf16 = jnp.bfloat16
    w128 = jnp.concatenate([wt[0:D] + wt[8 * D:9 * D], wt[12 * D:13 * D]],
                           axis=0).astype(bf16)
    wtd = (wt[4 * D:8 * D] - wt[8 * D:12 * D]).astype(bf16)

    din = _din(e0_g, er_g, seq_g, seq_mask, w128, wtd,
               fc1_b.reshape(1, -1), fc2_W, fc2_b.reshape(1, 1))
    return _cross(e0_g, er_g, din, ck.astype(bf16),
                  cb[0:1, :TOTP], cb[1:2, :TOTP],
                  d1_W.T.astype(bf16), d1_b.reshape(1, -1),
                  d2_W.T.astype(bf16), d2_b.reshape(1, -1),
                  lin_W[:, :TOTP], lin_W[:, F * D + 4 * D:],
                  lin_b.reshape(1, 1))


# speedup vs baseline: 3.1171x; 1.0123x over previous
import functools

import jax
import jax.numpy as jnp
from jax import lax
from jax.experimental import pallas as pl
from jax.experimental.pallas import tpu as pltpu
from jax.experimental.pallas import tpu_sc as plsc

B = 1024
F = 26
L = 50
D = 64
NF = F - 1
VREST = 1001
TOTX = F * D + D
TOTP = 1792
BB = 256
NC = 2
NS = 16
NW = NC * NS

PW_E0 = B // NW
PW_ER = (B * NF) // NW
PW_SEQ = (B * L) // NW
CH = 80


def _gather_body(emb0_hbm, embr_hbm, idx0_hbm, idxr_hbm, idxs_hbm,
                 e0_out, er_out, seq_out, idx_v, rows_v, sem):
    wid = lax.axis_index("s") * NC + lax.axis_index("c")

    def phase(table, idxh, outh, per_w):
        base = wid * per_w
        pltpu.sync_copy(idxh.at[pl.ds(base, per_w)], idx_v.at[pl.ds(0, per_w)])
        nch = per_w // CH
        cps = []
        if nch == 0:
            cps.append(pltpu.async_copy(
                table.at[idx_v.at[pl.ds(0, per_w)]],
                rows_v.at[pl.ds(0, per_w)], sem))
        else:
            for j in range(nch):
                cps.append(pltpu.async_copy(
                    table.at[idx_v.at[pl.ds(j * CH, CH)]],
                    rows_v.at[pl.ds(j * CH, CH)], sem))
        for cp in cps:
            cp.wait()
        pltpu.sync_copy(rows_v.at[pl.ds(0, per_w)], outh.at[pl.ds(base, per_w)])

    phase(emb0_hbm, idx0_hbm, e0_out, PW_E0)
    phase(embr_hbm, idxr_hbm, er_out, PW_ER)
    phase(emb0_hbm, idxs_hbm, seq_out, PW_SEQ)


def _sc_gather(emb0, embr, idx0, idxr, idxs):
    mesh = plsc.VectorSubcoreMesh(core_axis_name="c", subcore_axis_name="s")
    k = functools.partial(
        pl.kernel,
        mesh=mesh,
        compiler_params=pltpu.CompilerParams(use_tc_tiling_on_sc=False),
        out_type=[
            jax.ShapeDtypeStruct((B, D), jnp.float32),
            jax.ShapeDtypeStruct((B * NF, D), jnp.float32),
            jax.ShapeDtypeStruct((B * L, D), jnp.float32),
        ],
        scratch_types=[
            pltpu.VMEM((PW_SEQ,), jnp.int32),
            pltpu.VMEM((PW_SEQ, D), jnp.float32),
            pltpu.SemaphoreType.DMA,
        ],
    )(_gather_body)
    return k(emb0, embr, idx0, idxr, idxs)


def _din_body(e0_r, er_r, seq_r, mask_r, w128_r, wtd_r, fc1b_r, v2_r,
              fc2b_r, din_r):
    f32 = jnp.float32
    bf16 = jnp.bfloat16
    e0 = e0_r[...]
    seq2f = seq_r[...]
    seq = seq2f.reshape(BB, L, D)
    target = jnp.concatenate([e0, er_r[...][:, : 3 * D]], axis=1)
    c = jnp.dot(target.astype(bf16), wtd_r[...],
                preferred_element_type=f32) + fc1b_r[...]

    seq2 = seq2f
    su = (seq * e0[:, None, :]).reshape(BB * L, D)
    hm = (jnp.dot(seq2.astype(bf16), w128_r[...][:D],
                  preferred_element_type=f32)
          + jnp.dot(su.astype(bf16), w128_r[...][D:],
                    preferred_element_type=f32))
    h = jnp.maximum(hm.reshape(BB, L, 4 * D) + c[:, None, :], 0.0)
    awc = jnp.dot(h.reshape(BB * L, 4 * D).astype(bf16), v2_r[...],
                  preferred_element_type=f32)
    aw = awc.reshape(BB, L) + fc2b_r[...]
    aw = jnp.where(mask_r[...] < 0.5, -1e9, aw)
    m = jnp.max(aw, axis=1, keepdims=True)
    ex = jnp.exp(aw - m)
    p = ex / jnp.sum(ex, axis=1, keepdims=True)
    din_r[...] = (p[:, :, None] * seq).sum(1)


def _din(e0_g, er_g, seq_g, seq_mask, w128, wtd, fc1b, v2, fc2b):
    bi = lambda i: (i, 0)
    w2 = lambda i: (0, 0)
    in_specs = [
        pl.BlockSpec((BB, D), bi),
        pl.BlockSpec((BB, 4 * D), bi),
        pl.BlockSpec((BB * L, D), bi),
        pl.BlockSpec((BB, L), bi),
        pl.BlockSpec((2 * D, 4 * D), w2),
        pl.BlockSpec((4 * D, 4 * D), w2),
        pl.BlockSpec((1, 4 * D), w2),
        pl.BlockSpec((4 * D, 1), w2),
        pl.BlockSpec((1, 1), w2),
    ]
    return pl.pallas_call(
        _din_body,
        grid=(B // BB,),
        in_specs=in_specs,
        out_specs=pl.BlockSpec((BB, D), bi),
        out_shape=jax.ShapeDtypeStruct((B, D), jnp.float32),
        compiler_params=pltpu.CompilerParams(
            vmem_limit_bytes=100 * 1024 * 1024),
    )(e0_g, er_g, seq_g, seq_mask, w128, wtd, fc1b, v2, fc2b)


def _cross_body(e0_r, er_r, din_r, ck_r, cb0_r, cb1_r, d1w_r, d1b_r, d2w_r,
                d2b_r, linx_r, lind_r, linb_r, out_r):
    f32 = jnp.float32
    bf16 = jnp.bfloat16
    zer = jnp.zeros((B, TOTP - TOTX), f32)
    tx = jnp.concatenate([e0_r[...], er_r[...], din_r[...], zer], axis=1)
    txh = tx.astype(bf16)
    w0 = jnp.dot(txh, ck_r[0], preferred_element_type=f32) + cb0_r[...]
    x1 = tx * w0 + tx
    w1 = jnp.dot(x1.astype(bf16), ck_r[1],
                 preferred_element_type=f32) + cb1_r[...]
    x2 = tx * w1 + x1
    dh = jnp.maximum(
        lax.dot_general(txh, d1w_r[...], (((1,), (1,)), ((), ())),
                        preferred_element_type=f32) + d1b_r[...], 0.0)
    dh2 = jnp.maximum(
        lax.dot_general(dh.astype(bf16), d2w_r[...], (((1,), (1,)), ((), ())),
                        preferred_element_type=f32) + d2b_r[...], 0.0)
    out = ((x2 * linx_r[...]).sum(-1, keepdims=True)
           + (dh2 * lind_r[...]).sum(-1, keepdims=True) + linb_r[...])
    out_r[...] = out


def _cross(e0_g, er_g, din, ckh, cb0, cb1, d1w, d1b, d2w, d2b, linx, lind,
           linb):
    w2 = lambda i: (0, 0)
    in_specs = [
        pl.BlockSpec((B, D), w2),
        pl.BlockSpec((B, NF * D), w2),
        pl.BlockSpec((B, D), w2),
        pl.BlockSpec((2, TOTP, TOTP), lambda i: (0, 0, 0)),
        pl.BlockSpec((1, TOTP), w2),
        pl.BlockSpec((1, TOTP), w2),
        pl.BlockSpec((1024, TOTP), w2),
        pl.BlockSpec((1, 1024), w2),
        pl.BlockSpec((512, 1024), w2),
        pl.BlockSpec((1, 512), w2),
        pl.BlockSpec((1, TOTP), w2),
        pl.BlockSpec((1, 512), w2),
        pl.BlockSpec((1, 1), w2),
    ]
    return pl.pallas_call(
        _cross_body,
        grid=(1,),
        in_specs=in_specs,
        out_specs=pl.BlockSpec((B, 1), w2),
        out_shape=jax.ShapeDtypeStruct((B, 1), jnp.float32),
        compiler_params=pltpu.CompilerParams(
            vmem_limit_bytes=120 * 1024 * 1024),
    )(e0_g, er_g, din, ckh, cb0, cb1, d1w, d1b, d2w, d2b, linx, lind, linb)


def kernel(dnn_feat, seq_feat, seq_mask, emb0, emb_rest, fc1_W, fc1_b, fc2_W,
           fc2_b, ck, cb, d1_W, d1_b, d2_W, d2_b, lin_W, lin_b):
    dnn_feat = dnn_feat.astype(jnp.int32)
    seq_feat = seq_feat.astype(jnp.int32)
    embr = emb_rest.reshape(NF * VREST, D)
    idx0 = dnn_feat[:, 0]
    offs = (jnp.arange(NF, dtype=jnp.int32) * VREST)[None, :]
    idxr = (dnn_feat[:, 1:] + offs).reshape(-1)
    idxs = seq_feat.reshape(-1)

    e0_g, er_g, seq_g = _sc_gather(emb0, embr, idx0, idxr, idxs)
    er_g = er_g.reshape(B, NF * D)

    wt = fc1_W.T
    bf16 = jnp.bfloat16
    w128 = jnp.concatenate([wt[0:D] + wt[8 * D:9 * D], wt[12 * D:13 * D]],
                           axis=0).astype(bf16)
    wtd = (wt[4 * D:8 * D] - wt[8 * D:12 * D]).astype(bf16)

    din = _din(e0_g, er_g, seq_g, seq_mask, w128, wtd,
               fc1_b.reshape(1, -1), fc2_W.T.astype(bf16),
               fc2_b.reshape(1, 1))
    return _cross(e0_g, er_g, din, ck.astype(bf16),
                  cb[0:1, :TOTP], cb[1:2, :TOTP],
                  d1_W.astype(bf16), d1_b.reshape(1, -1),
                  d2_W.astype(bf16), d2_b.reshape(1, -1),
                  lin_W[:, :TOTP], lin_W[:, F * D + 4 * D:],
                  lin_b.reshape(1, 1))


# speedup vs baseline: 3.3138x; 1.0631x over previous
import functools

import jax
import jax.numpy as jnp
from jax import lax
from jax.experimental import pallas as pl
from jax.experimental.pallas import tpu as pltpu
from jax.experimental.pallas import tpu_sc as plsc

B = 1024
F = 26
L = 50
D = 64
NF = F - 1
VREST = 1001
TOTX = F * D + D
TOTP = 1792
BB = 256
NC = 2
NS = 16
NW = NC * NS

PW_E0 = B // NW
PW_ER = (B * NF) // NW
PW_SEQ = (B * L) // NW
CH = 80


def _gather_body(emb0_hbm, embr_hbm, idx0_hbm, idxr_hbm, idxs_hbm,
                 e0_out, er_out, seq_out, idx_v, rows_v, sem):
    wid = lax.axis_index("s") * NC + lax.axis_index("c")

    def phase(table, idxh, outh, per_w):
        base = wid * per_w
        pltpu.sync_copy(idxh.at[pl.ds(base, per_w)], idx_v.at[pl.ds(0, per_w)])
        nch = per_w // CH
        cps = []
        if nch == 0:
            cps.append(pltpu.async_copy(
                table.at[idx_v.at[pl.ds(0, per_w)]],
                rows_v.at[pl.ds(0, per_w)], sem))
        else:
            for j in range(nch):
                cps.append(pltpu.async_copy(
                    table.at[idx_v.at[pl.ds(j * CH, CH)]],
                    rows_v.at[pl.ds(j * CH, CH)], sem))
        for cp in cps:
            cp.wait()
        pltpu.sync_copy(rows_v.at[pl.ds(0, per_w)], outh.at[pl.ds(base, per_w)])

    phase(emb0_hbm, idx0_hbm, e0_out, PW_E0)
    phase(embr_hbm, idxr_hbm, er_out, PW_ER)
    phase(emb0_hbm, idxs_hbm, seq_out, PW_SEQ)


def _sc_gather(emb0, embr, idx0, idxr, idxs):
    mesh = plsc.VectorSubcoreMesh(core_axis_name="c", subcore_axis_name="s")
    k = functools.partial(
        pl.kernel,
        mesh=mesh,
        compiler_params=pltpu.CompilerParams(use_tc_tiling_on_sc=False),
        out_type=[
            jax.ShapeDtypeStruct((B, D), jnp.float32),
            jax.ShapeDtypeStruct((B * NF, D), jnp.float32),
            jax.ShapeDtypeStruct((B * L, D), jnp.float32),
        ],
        scratch_types=[
            pltpu.VMEM((PW_SEQ,), jnp.int32),
            pltpu.VMEM((PW_SEQ, D), jnp.float32),
            pltpu.SemaphoreType.DMA,
        ],
    )(_gather_body)
    return k(emb0, embr, idx0, idxr, idxs)


def _din_body(e0_r, er_r, seq_r, mask_r, w128_r, wtd_r, fc1b_r, v2_r,
              fc2b_r, din_r):
    f32 = jnp.float32
    bf16 = jnp.bfloat16
    e0 = e0_r[...]
    seq2f = seq_r[...]
    seq = seq2f.reshape(BB, L, D)
    target = jnp.concatenate([e0, er_r[...][:, : 3 * D]], axis=1)
    c = jnp.dot(target.astype(bf16), wtd_r[...],
                preferred_element_type=f32) + fc1b_r[...]

    seq2 = seq2f
    su = (seq * e0[:, None, :]).reshape(BB * L, D)
    hm = (jnp.dot(seq2.astype(bf16), w128_r[...][:D],
                  preferred_element_type=f32)
          + jnp.dot(su.astype(bf16), w128_r[...][D:],
                    preferred_element_type=f32))
    h = jnp.maximum(hm.reshape(BB, L, 4 * D) + c[:, None, :], 0.0)
    awc = jnp.dot(h.reshape(BB * L, 4 * D).astype(bf16), v2_r[...],
                  preferred_element_type=f32)
    aw = awc.reshape(BB, L) + fc2b_r[...]
    aw = jnp.where(mask_r[...] < 0.5, -1e9, aw)
    m = jnp.max(aw, axis=1, keepdims=True)
    ex = jnp.exp(aw - m)
    p = ex / jnp.sum(ex, axis=1, keepdims=True)
    din_r[...] = (p[:, :, None] * seq).sum(1)


def _din(e0_g, er_g, seq_g, seq_mask, w128, wtd, fc1b, v2, fc2b):
    bi = lambda i: (i, 0)
    w2 = lambda i: (0, 0)
    in_specs = [
        pl.BlockSpec((BB, D), bi),
        pl.BlockSpec((BB, 4 * D), bi),
        pl.BlockSpec((BB * L, D), bi),
        pl.BlockSpec((BB, L), bi),
        pl.BlockSpec((2 * D, 4 * D), w2),
        pl.BlockSpec((4 * D, 4 * D), w2),
        pl.BlockSpec((1, 4 * D), w2),
        pl.BlockSpec((4 * D, 1), w2),
        pl.BlockSpec((1, 1), w2),
    ]
    return pl.pallas_call(
        _din_body,
        grid=(B // BB,),
        in_specs=in_specs,
        out_specs=pl.BlockSpec((BB, D), bi),
        out_shape=jax.ShapeDtypeStruct((B, D), jnp.float32),
        compiler_params=pltpu.CompilerParams(
            vmem_limit_bytes=100 * 1024 * 1024),
    )(e0_g, er_g, seq_g, seq_mask, w128, wtd, fc1b, v2, fc2b)


def _cross_body(e0_r, er_r, din_r, ck_r, cb0_r, cb1_r, d1w_r, d1b_r, d2w_r,
                d2b_r, linx_r, lind_r, linb_r, out_r):
    f32 = jnp.float32
    bf16 = jnp.bfloat16
    zer = jnp.zeros((B, TOTP - TOTX), f32)
    tx = jnp.concatenate([e0_r[...], er_r[...], din_r[...], zer], axis=1)
    txh = tx.astype(bf16)
    ck0h = ck_r[0].astype(bf16)
    ck1h = ck_r[1].astype(bf16)
    w0 = jnp.dot(txh, ck0h, preferred_element_type=f32) + cb0_r[...]
    x1 = tx * w0 + tx
    w1 = jnp.dot(x1.astype(bf16), ck1h,
                 preferred_element_type=f32) + cb1_r[...]
    x2 = tx * w1 + x1
    dh = jnp.maximum(
        lax.dot_general(txh, d1w_r[...].astype(bf16),
                        (((1,), (1,)), ((), ())),
                        preferred_element_type=f32) + d1b_r[...], 0.0)
    dh2 = jnp.maximum(
        lax.dot_general(dh.astype(bf16), d2w_r[...].astype(bf16),
                        (((1,), (1,)), ((), ())),
                        preferred_element_type=f32) + d2b_r[...], 0.0)
    out = ((x2 * linx_r[...]).sum(-1, keepdims=True)
           + (dh2 * lind_r[...]).sum(-1, keepdims=True) + linb_r[...])
    out_r[...] = out


def _cross(e0_g, er_g, din, ckh, cb0, cb1, d1w, d1b, d2w, d2b, linx, lind,
           linb):
    w2 = lambda i: (0, 0)
    in_specs = [
        pl.BlockSpec((B, D), w2),
        pl.BlockSpec((B, NF * D), w2),
        pl.BlockSpec((B, D), w2),
        pl.BlockSpec((2, TOTP, TOTP), lambda i: (0, 0, 0)),
        pl.BlockSpec((1, TOTP), w2),
        pl.BlockSpec((1, TOTP), w2),
        pl.BlockSpec((1024, TOTP), w2),
        pl.BlockSpec((1, 1024), w2),
        pl.BlockSpec((512, 1024), w2),
        pl.BlockSpec((1, 512), w2),
        pl.BlockSpec((1, TOTP), w2),
        pl.BlockSpec((1, 512), w2),
        pl.BlockSpec((1, 1), w2),
    ]
    return pl.pallas_call(
        _cross_body,
        grid=(1,),
        in_specs=in_specs,
        out_specs=pl.BlockSpec((B, 1), w2),
        out_shape=jax.ShapeDtypeStruct((B, 1), jnp.float32),
        compiler_params=pltpu.CompilerParams(
            vmem_limit_bytes=120 * 1024 * 1024),
    )(e0_g, er_g, din, ckh, cb0, cb1, d1w, d1b, d2w, d2b, linx, lind, linb)


def kernel(dnn_feat, seq_feat, seq_mask, emb0, emb_rest, fc1_W, fc1_b, fc2_W,
           fc2_b, ck, cb, d1_W, d1_b, d2_W, d2_b, lin_W, lin_b):
    dnn_feat = dnn_feat.astype(jnp.int32)
    seq_feat = seq_feat.astype(jnp.int32)
    embr = emb_rest.reshape(NF * VREST, D)
    idx0 = dnn_feat[:, 0]
    offs = (jnp.arange(NF, dtype=jnp.int32) * VREST)[None, :]
    idxr = (dnn_feat[:, 1:] + offs).reshape(-1)
    idxs = seq_feat.reshape(-1)

    e0_g, er_g, seq_g = _sc_gather(emb0, embr, idx0, idxr, idxs)
    er_g = er_g.reshape(B, NF * D)

    wt = fc1_W.T
    bf16 = jnp.bfloat16
    w128 = jnp.concatenate([wt[0:D] + wt[8 * D:9 * D], wt[12 * D:13 * D]],
                           axis=0).astype(bf16)
    wtd = (wt[4 * D:8 * D] - wt[8 * D:12 * D]).astype(bf16)

    din = _din(e0_g, er_g, seq_g, seq_mask, w128, wtd,
               fc1_b.reshape(1, -1), fc2_W.T.astype(bf16),
               fc2_b.reshape(1, 1))
    return _cross(e0_g, er_g, din, ck,
                  cb[0:1, :TOTP], cb[1:2, :TOTP],
                  d1_W, d1_b.reshape(1, -1),
                  d2_W, d2_b.reshape(1, -1),
                  lin_W[:, :TOTP], lin_W[:, F * D + 4 * D:],
                  lin_b.reshape(1, 1))
